# Initial kernel scaffold; baseline (speedup 1.0000x reference)
import jax
import jax.numpy as jnp
from jax.experimental import pallas as pl


def kernel(x, edge_index, batch, W0, b0, Wa0, ba0, Wb0, bb0, Wroot0, Wrel0, brel0, Wa1, ba1, Wb1, bb1, Wroot1, Wrel1, brel1, Wlin, blin, Wr1, br1, g1, bt1, Wr2, br2, g2, bt2, Wr3):
    raise NotImplementedError("write your pallas kernel here")


# speedup vs baseline: 8.8585x; 8.8585x over previous
import functools

import jax
import jax.numpy as jnp
import numpy as np
from jax import lax
from jax.experimental import pallas as pl
from jax.experimental.pallas import tpu as pltpu
from jax.experimental.pallas import tpu_sc as plsc

N = 10000
NP = 10240
NF = NP // 128
D = 128
E = 320000
NT = 32
EPT = 10112
CH = 128
NCH = EPT // CH
RPT = NP // 16
K0 = 6000
K1 = 3600
INT_MIN = np.int32(-2147483648)
HI_P = lax.Precision.HIGHEST


@functools.cache
def _build_segsum_rows():
    mesh = plsc.VectorSubcoreMesh(core_axis_name="c", subcore_axis_name="s")

    @functools.partial(
        pl.kernel,
        mesh=mesh,
        out_type=jax.ShapeDtypeStruct((2 * NP, D), jnp.float32),
        scratch_types=[
            pltpu.VMEM((CH,), jnp.int32),
            pltpu.VMEM((CH,), jnp.int32),
            pltpu.VMEM((CH, D), jnp.float32),
            pltpu.VMEM_SHARED((NP, D), jnp.float32),
            pltpu.SemaphoreType.DMA,
        ],
    )
    def segsum_rows(h_hbm, src_hbm, dst_hbm, zeros_hbm, out_hbm,
                    sidx, didx, rows, acc, sem):
        c = lax.axis_index("c")
        s = lax.axis_index("s")
        wid = s * 2 + c
        base = wid * EPT
        pltpu.sync_copy(zeros_hbm.at[pl.ds(s * RPT, RPT), :],
                        acc.at[pl.ds(s * RPT, RPT), :])
        plsc.subcore_barrier()

        def body(i, carry):
            off = base + i * CH
            pltpu.sync_copy(src_hbm.at[pl.ds(off, CH)], sidx)
            pltpu.sync_copy(dst_hbm.at[pl.ds(off, CH)], didx)
            pltpu.async_copy(h_hbm.at[sidx], rows, sem).wait()
            pltpu.sync_copy(rows, acc.at[didx], add=True)
            return carry

        lax.fori_loop(0, NCH, body, 0)
        plsc.subcore_barrier()
        pltpu.sync_copy(acc.at[pl.ds(s * RPT, RPT), :],
                        out_hbm.at[pl.ds(c * NP + s * RPT, RPT), :])

    return segsum_rows


@functools.cache
def _build_segsum_scal():
    mesh = plsc.VectorSubcoreMesh(core_axis_name="c", subcore_axis_name="s")

    @functools.partial(
        pl.kernel,
        mesh=mesh,
        out_type=jax.ShapeDtypeStruct((2, NP), jnp.float32),
        scratch_types=[
            pltpu.VMEM((CH,), jnp.int32),
            pltpu.VMEM((CH,), jnp.int32),
            pltpu.VMEM((CH,), jnp.float32),
            pltpu.VMEM_SHARED((NP,), jnp.float32),
            pltpu.SemaphoreType.DMA,
        ],
    )
    def segsum_scal(v_hbm, src_hbm, dst_hbm, zeros_hbm, out_hbm,
                    sidx, didx, vals, acc, sem):
        c = lax.axis_index("c")
        s = lax.axis_index("s")
        wid = s * 2 + c
        base = wid * EPT
        pltpu.sync_copy(zeros_hbm.at[pl.ds(s * RPT, RPT)],
                        acc.at[pl.ds(s * RPT, RPT)])
        plsc.subcore_barrier()

        def body(i, carry):
            off = base + i * CH
            pltpu.sync_copy(src_hbm.at[pl.ds(off, CH)], sidx)
            pltpu.sync_copy(dst_hbm.at[pl.ds(off, CH)], didx)
            pltpu.async_copy(v_hbm.at[sidx], vals, sem).wait()
            pltpu.sync_copy(vals, acc.at[didx], add=True)
            return carry

        lax.fori_loop(0, NCH, body, 0)
        plsc.subcore_barrier()
        pltpu.sync_copy(acc.at[pl.ds(s * RPT, RPT)],
                        out_hbm.at[c, pl.ds(s * RPT, RPT)])

    return segsum_scal


def _dot(a, b):
    return jnp.dot(a, b, preferred_element_type=jnp.float32, precision=HI_P)


def _first_h(x_ref, w_ref, b_ref, out_ref):
    h = jnp.maximum(_dot(x_ref[...], w_ref[...]) + b_ref[...], 0.0)
    out_ref[0:N, :] = h
    out_ref[N:NP, :] = jnp.zeros((NP - N, D), jnp.float32)


RB = 1280
GB = NP // RB


def _gin_mlp(h_ref, a0_ref, a1_ref, alive_ref, wa_ref, ba_ref, wb_ref,
             bb_ref, wroot_ref, wrel_ref, z_ref, zroot_ref, zrm_ref):
    t = h_ref[...] + a0_ref[...] + a1_ref[...]
    u = jnp.maximum(_dot(t, wa_ref[...]) + ba_ref[...], 0.0)
    z = _dot(u, wb_ref[...]) + bb_ref[...]
    z_ref[...] = z
    zroot_ref[...] = _dot(z, wroot_ref[...])
    zrm_ref[...] = _dot(z, wrel_ref[...]) * alive_ref[...]


def _gin_mlp_call(h, a0, a1, alive_c, Wa, ba, Wb, bb, Wroot, Wrel):
    rows = lambda: pl.BlockSpec((RB, D), lambda i: (i, 0))
    col = lambda: pl.BlockSpec((RB, 1), lambda i: (i, 0))
    full = lambda a, b: pl.BlockSpec((a, b), lambda i: (0, 0))
    return pl.pallas_call(
        _gin_mlp,
        grid=(GB,),
        in_specs=[rows(), rows(), rows(), col(),
                  full(D, D), full(1, D), full(D, D), full(1, D),
                  full(D, 1), full(D, 1)],
        out_specs=[rows(), col(), col()],
        out_shape=[jax.ShapeDtypeStruct((NP, D), jnp.float32),
                   jax.ShapeDtypeStruct((NP, 1), jnp.float32),
                   jax.ShapeDtypeStruct((NP, 1), jnp.float32)],
    )(h, a0, a1, alive_c, Wa, ba, Wb, bb, Wroot, Wrel)


def _sortable_key(s):
    b = lax.bitcast_convert_type(s, jnp.int32)
    return jnp.where(b < 0, b ^ np.int32(0x7FFFFFFF), b)


def _select_topk(km_f, k):
    def body(i, lohi):
        lo, hi = lohi
        x = lo ^ hi
        mid = (lo & hi) + (x >> 1) + (x & 1)
        cnt = jnp.sum((km_f >= mid).astype(jnp.int32))
        ok = cnt >= k
        return (jnp.where(ok, mid, lo), jnp.where(ok, hi, mid - 1))

    lo, hi = lax.fori_loop(0, 32, body, (INT_MIN, np.int32(2**31 - 1)))
    t = lo
    need = k - jnp.sum((km_f > t).astype(jnp.int32))
    eq = km_f == t
    fidx = (lax.broadcasted_iota(jnp.int32, (NF, 128), 0) * 128
            + lax.broadcasted_iota(jnp.int32, (NF, 128), 1))

    def body2(i, lohi):
        lo2, hi2 = lohi
        mid = (lo2 + hi2) // 2
        cnt = jnp.sum((eq & (fidx < mid)).astype(jnp.int32))
        ok = cnt >= need
        return (jnp.where(ok, lo2, mid + 1), jnp.where(ok, mid, hi2))

    lo2, _ = lax.fori_loop(0, 15, body2, (np.int32(0), np.int32(NP)))
    return t, lo2


def _make_pool_thresh(k):
    def pool_thresh(ss0_f, ss1_f, zroot_f, alive_f, brel,
                    t_ref, mstar_ref, nm_f_ref):
        s_f = ss0_f[...] + ss1_f[...] + zroot_f[...] + brel[0, 0]
        fidx = (lax.broadcasted_iota(jnp.int32, (NF, 128), 0) * 128
                + lax.broadcasted_iota(jnp.int32, (NF, 128), 1))
        valid_f = (alive_f[...] > 0.0) & (fidx < N)
        km_f = jnp.where(valid_f, _sortable_key(s_f), INT_MIN)
        t, mstar = _select_topk(km_f, k)
        t_ref[...] = jnp.broadcast_to(t, (1, 1))
        mstar_ref[...] = jnp.broadcast_to(mstar, (1, 1))
        nm_f = (km_f > t) | ((km_f == t) & (fidx < mstar))
        nm_f_ref[...] = nm_f.astype(jnp.float32)

    return pl.pallas_call(
        pool_thresh,
        out_shape=[jax.ShapeDtypeStruct((1, 1), jnp.int32),
                   jax.ShapeDtypeStruct((1, 1), jnp.int32),
                   jax.ShapeDtypeStruct((NF, 128), jnp.float32)])


def _pool_apply(z_ref, zroot_c, ss0_c, ss1_c, alive_c, brel, t_ref,
                mstar_ref, h_ref, nm_c_ref, pooled_ref, *, inv_k):
    i = pl.program_id(0)
    s_c = ss0_c[...] + ss1_c[...] + zroot_c[...] + brel[0, 0]
    cidx = i * RB + lax.broadcasted_iota(jnp.int32, (RB, 1), 0)
    valid_c = (alive_c[...] > 0.0) & (cidx < N)
    km_c = jnp.where(valid_c, _sortable_key(s_c), INT_MIN)
    t = t_ref[0, 0]
    mstar = mstar_ref[0, 0]
    nm_c = (km_c > t) | ((km_c == t) & (cidx < mstar))
    factor = jnp.tanh(s_c) * nm_c.astype(jnp.float32)
    h_new = z_ref[...] * factor
    h_ref[...] = h_new
    nm_c_ref[...] = nm_c.astype(jnp.float32)

    @pl.when(i == 0)
    def _():
        pooled_ref[...] = jnp.zeros((1, D), jnp.float32)

    pooled_ref[...] += jnp.sum(h_new, axis=0, keepdims=True) * inv_k


def _pool_apply_call(z, zroot_c, ss0_c, ss1_c, alive_c, brel, t, mstar, k):
    rows = lambda: pl.BlockSpec((RB, D), lambda i: (i, 0))
    col = lambda: pl.BlockSpec((RB, 1), lambda i: (i, 0))
    one = lambda: pl.BlockSpec((1, 1), lambda i: (0, 0))
    return pl.pallas_call(
        functools.partial(_pool_apply, inv_k=np.float32(1.0 / k)),
        grid=(GB,),
        in_specs=[rows(), col(), col(), col(), col(), one(), one(), one()],
        out_specs=[rows(), col(), pl.BlockSpec((1, D), lambda i: (0, 0))],
        out_shape=[jax.ShapeDtypeStruct((NP, D), jnp.float32),
                   jax.ShapeDtypeStruct((NP, 1), jnp.float32),
                   jax.ShapeDtypeStruct((1, D), jnp.float32)],
    )(z, zroot_c, ss0_c, ss1_c, alive_c, brel, t, mstar)


def _head(pooled0_ref, pooled1_ref, wlin_ref, blin_ref, wr1_ref, br1_ref,
          g1_ref, bt1_ref, wr2_ref, br2_ref, g2_ref, bt2_ref, wr3_ref,
          out_ref):
    g = jnp.concatenate([pooled0_ref[...], pooled1_ref[...]], axis=1)
    out = _dot(g, wlin_ref[...]) + blin_ref[...]
    scale = np.float32(1.0 / np.sqrt(1.0 + 1e-5))
    h1 = jnp.maximum((_dot(out, wr1_ref[...]) + br1_ref[...]) * scale
                     * g1_ref[...] + bt1_ref[...], 0.0)
    h2 = jnp.maximum((_dot(h1, wr2_ref[...]) + br2_ref[...]) * scale
                     * g2_ref[...] + bt2_ref[...], 0.0)
    out_ref[...] = _dot(h2, wr3_ref[...])


def _tc_call(body, out_shapes):
    return pl.pallas_call(
        body,
        out_shape=[jax.ShapeDtypeStruct(s, jnp.float32) for s in out_shapes])


def kernel(x, edge_index, batch, W0, b0, Wa0, ba0, Wb0, bb0, Wroot0, Wrel0,
           brel0, Wa1, ba1, Wb1, bb1, Wroot1, Wrel1, brel1, Wlin, blin,
           Wr1, br1, g1, bt1, Wr2, br2, g2, bt2, Wr3):
    f32 = jnp.float32
    src2 = edge_index[0].reshape(NT, E // NT)
    dst2 = edge_index[1].reshape(NT, E // NT)
    pad = jnp.full((NT, EPT - E // NT), N, jnp.int32)
    src_p = jnp.concatenate([src2, pad], axis=1).reshape(-1)
    dst_p = jnp.concatenate([dst2, pad], axis=1).reshape(-1)
    zeros_rows = jnp.zeros((NP, D), f32)
    zeros_vec = jnp.zeros((NP,), f32)
    alive_c = jnp.where(jnp.arange(NP)[:, None] < N, 1.0, 0.0).astype(f32)
    alive_f = alive_c.reshape(NF, 128)

    row = lambda v: v.reshape(1, -1)

    (h,) = _tc_call(_first_h, [(NP, D)])(x, W0, row(b0))

    params = ((Wa0, ba0, Wb0, bb0, Wroot0, Wrel0, brel0, K0),
              (Wa1, ba1, Wb1, bb1, Wroot1, Wrel1, brel1, K1))
    pooled = []
    for l in range(2):
        Wa, ba, Wb, bb, Wroot, Wrel, brel, k = params[l]
        agg2 = _build_segsum_rows()(h, src_p, dst_p, zeros_rows)
        z, zroot_c, zrm_c = _gin_mlp_call(
            h, agg2[0:NP], agg2[NP:2 * NP], alive_c,
            Wa, row(ba), Wb, row(bb), Wroot, Wrel)
        ss2 = _build_segsum_scal()(zrm_c.reshape(NP), src_p, dst_p, zeros_vec)
        ss0_c, ss1_c = ss2[0].reshape(NP, 1), ss2[1].reshape(NP, 1)
        ss0_f, ss1_f = ss2[0].reshape(NF, 128), ss2[1].reshape(NF, 128)
        zroot_f = zroot_c.reshape(NF, 128)
        brel_a = brel.reshape(1, 1)
        t, mstar, nm_f = _make_pool_thresh(k)(
            ss0_f, ss1_f, zroot_f, alive_f, brel_a)
        h, alive_c, pl_l = _pool_apply_call(
            z, zroot_c, ss0_c, ss1_c, alive_c, brel_a, t, mstar, k)
        alive_f = nm_f
        pooled.append(pl_l)
    (out,) = _tc_call(_head, [(1, 4)])(
        pooled[0], pooled[1],
        Wlin, row(blin), Wr1, row(br1), row(g1), row(bt1),
        Wr2, row(br2), row(g2), row(bt2), Wr3)
    return out
